# baseline (speedup 1.0000x reference)
import functools

import jax
import jax.numpy as jnp
from jax import lax
from jax.experimental import pallas as pl
from jax.experimental.pallas import tpu as pltpu
from jax.experimental.pallas import tpu_sc as plsc

VOCAB = 1000000
EMBED = 64
BATCH = 16384

_info = plsc.get_sparse_core_info()
NC, NS, L = _info.num_cores, _info.num_subcores, _info.num_lanes
NW = NC * NS
BPW = BATCH // NW
CHUNK = 128
NCHUNK = BPW // CHUNK
NGROUP = BPW // 16


def _sc_body(x_hbm, t_hbm, tgt_hbm, ctx_hbm, out_hbm,
             xidx, tidx, tbuf, cbuf, outv, semt, semc):
    wid = lax.axis_index("s") * NC + lax.axis_index("c")
    base = wid * BPW

    pltpu.sync_copy(x_hbm.at[pl.ds(wid * NCHUNK, NCHUNK)], xidx)
    pltpu.sync_copy(t_hbm.at[pl.ds(wid * NCHUNK, NCHUNK)], tidx)

    copies = []
    for j in range(NCHUNK):
        copies.append(pltpu.async_copy(
            tgt_hbm.at[tidx.at[j]], tbuf.at[pl.ds(j * CHUNK, CHUNK)], semt))
        copies.append(pltpu.async_copy(
            ctx_hbm.at[xidx.at[j]], cbuf.at[pl.ds(j * CHUNK, CHUNK)], semc))
    for c in copies:
        c.wait()

    lane = lax.iota(jnp.int32, L)

    def group(g, carry):
        rows = g * 16 + lane
        acc = jnp.zeros((L,), jnp.float32)
        for d in range(EMBED):
            col = jnp.full((L,), d, jnp.int32)
            tv = plsc.load_gather(tbuf, [rows, col])
            cv = plsc.load_gather(cbuf, [rows, col])
            acc = acc + tv * cv
        outv[pl.ds(g * 16, 16)] = acc
        return carry

    lax.fori_loop(0, NGROUP, group, 0)
    pltpu.sync_copy(outv, out_hbm.at[pl.ds(base, BPW)])


@jax.jit
def _sc_call(x2d, t2d, target_table, context_table):
    mesh = plsc.VectorSubcoreMesh(core_axis_name="c", subcore_axis_name="s")
    return pl.kernel(
        _sc_body,
        out_type=jax.ShapeDtypeStruct((BATCH,), jnp.float32),
        mesh=mesh,
        compiler_params=pltpu.CompilerParams(
            needs_layout_passes=False,
            use_tc_tiling_on_sc=False,
        ),
        scratch_types=[
            pltpu.VMEM((NCHUNK, CHUNK), jnp.int32),
            pltpu.VMEM((NCHUNK, CHUNK), jnp.int32),
            pltpu.VMEM((BPW, EMBED), jnp.float32),
            pltpu.VMEM((BPW, EMBED), jnp.float32),
            pltpu.VMEM((BPW,), jnp.float32),
            pltpu.SemaphoreType.DMA,
            pltpu.SemaphoreType.DMA,
        ],
    )(x2d, t2d, target_table, context_table)


def kernel(x, t, target_table, context_table):
    x2d = x.reshape(NW * NCHUNK, CHUNK)
    t2d = t.reshape(NW * NCHUNK, CHUNK)
    return _sc_call(x2d, t2d, target_table, context_table)


# speedup vs baseline: 1.6048x; 1.6048x over previous
import jax
import jax.numpy as jnp
from jax import lax
from jax.experimental import pallas as pl
from jax.experimental.pallas import tpu as pltpu
from jax.experimental.pallas import tpu_sc as plsc

VOCAB = 1000000
EMBED = 64
BATCH = 16384

_info = plsc.get_sparse_core_info()
NC, NS, L = _info.num_cores, _info.num_subcores, _info.num_lanes
NW = NC * NS
BPW = BATCH // NW
CHUNK = 32
NCHUNK = BPW // CHUNK
NVREG = EMBED // 16


def _sc_body(x_hbm, t_hbm, tgt_hbm, ctx_hbm, out_hbm,
             xidx, tidx, tbuf, cbuf, outv, semt, semc):
    wid = lax.axis_index("s") * NC + lax.axis_index("c")
    base = wid * BPW

    pltpu.sync_copy(x_hbm.at[pl.ds(base, BPW)], xidx)
    pltpu.sync_copy(t_hbm.at[pl.ds(base, BPW)], tidx)

    lane = lax.iota(jnp.int32, L)

    def chunk_step(p, carry):
        cbase = p * CHUNK
        descs = []
        for cc in range(CHUNK // 16):
            tv = tidx[pl.ds(cbase + cc * 16, 16)]
            xv = xidx[pl.ds(cbase + cc * 16, 16)]
            for j in range(16):
                row = cc * 16 + j
                descs.append(pltpu.async_copy(
                    tgt_hbm.at[tv[j]], tbuf.at[row, pl.ds(0, EMBED)], semt))
                descs.append(pltpu.async_copy(
                    ctx_hbm.at[xv[j]], cbuf.at[row, pl.ds(0, EMBED)], semc))
        for d in descs:
            d.wait()
        for cc in range(CHUNK // 16):
            res = jnp.zeros((L,), jnp.float32)
            for j in range(16):
                row = cc * 16 + j
                s = jnp.zeros((L,), jnp.float32)
                for k in range(NVREG):
                    s = s + (tbuf[row, pl.ds(k * 16, 16)]
                             * cbuf[row, pl.ds(k * 16, 16)])
                tot = jnp.sum(s)
                res = jnp.where(lane == j, tot, res)
            outv[pl.ds(cbase + cc * 16, 16)] = res
        return carry

    lax.fori_loop(0, NCHUNK, chunk_step, 0)
    pltpu.sync_copy(outv, out_hbm.at[pl.ds(base, BPW)])


@jax.jit
def _sc_call(x, t, target_table, context_table):
    mesh = plsc.VectorSubcoreMesh(core_axis_name="c", subcore_axis_name="s")
    return pl.kernel(
        _sc_body,
        out_type=jax.ShapeDtypeStruct((BATCH,), jnp.float32),
        mesh=mesh,
        compiler_params=pltpu.CompilerParams(
            needs_layout_passes=False,
        ),
        scratch_types=[
            pltpu.VMEM((BPW,), jnp.int32),
            pltpu.VMEM((BPW,), jnp.int32),
            pltpu.VMEM((CHUNK, 2 * EMBED), jnp.float32),
            pltpu.VMEM((CHUNK, 2 * EMBED), jnp.float32),
            pltpu.VMEM((BPW,), jnp.float32),
            pltpu.SemaphoreType.DMA,
            pltpu.SemaphoreType.DMA,
        ],
    )(x, t, target_table, context_table)


def kernel(x, t, target_table, context_table):
    return _sc_call(x, t, target_table, context_table)
